# baseline (speedup 1.0000x reference)
import functools

import jax
import jax.numpy as jnp
import numpy as np
from jax import lax
from jax.experimental import pallas as pl
from jax.experimental.pallas import tpu as pltpu
from jax.experimental.pallas import tpu_sc as plsc

_MASK_RATIO = 0.15
_ONE_BITS = 0x3F800000
_L = 16

_CONST_CACHE = {}


def _fixed_tables(B, N):
    if (B, N) not in _CONST_CACHE:
        with jax.ensure_compile_time_eval():
            r = np.asarray(
                jax.random.uniform(jax.random.key(42), (B, N), dtype=jnp.float32))
        bits = r.view(np.int32)
        sidx = np.argsort(bits, axis=1).astype(np.int32)
        sbits = np.take_along_axis(bits, sidx, axis=1)
        _CONST_CACHE[(B, N)] = (bits, sidx.reshape(-1), sbits.reshape(-1))
    return _CONST_CACHE[(B, N)]


def _build_sc_select(B, N):
    NCH = N // _L
    mesh = plsc.VectorSubcoreMesh(core_axis_name="c", subcore_axis_name="s")

    @functools.partial(
        pl.kernel, mesh=mesh,
        compiler_params=pltpu.CompilerParams(needs_layout_passes=False),
        out_type=jax.ShapeDtypeStruct((B * N,), jnp.int32),
        scratch_types=[
            pltpu.VMEM((N,), jnp.int32),
            pltpu.VMEM((N,), jnp.int32),
            pltpu.VMEM((N,), jnp.int32),
            pltpu.VMEM((N,), jnp.int32),
            pltpu.VMEM((N,), jnp.int32),
            pltpu.VMEM((N,), jnp.int32),
        ],
    )
    def sel(pad_hbm, sidx_hbm, sbits_hbm, bits_hbm, mask_hbm,
            prow_v, pown_v, sidx_v, sbits_v, bits_v, mask_v):
        wid = lax.axis_index("s") * 2 + lax.axis_index("c")

        @pl.when(wid < B)
        def _():
            row = wid
            lanes = jax.lax.broadcasted_iota(jnp.int32, (_L,), 0)

            def count_row(r, tot):
                pltpu.sync_copy(pad_hbm.at[pl.ds(r * N, N)], prow_v)

                def cbody(i, a):
                    v = prow_v[pl.ds(i * _L, _L)]
                    return a + jnp.where(v == 0, 1, 0)

                acc = lax.fori_loop(0, NCH, cbody, jnp.zeros((_L,), jnp.int32))
                c0 = prow_v[pl.ds(0, _L)]
                corr = jnp.sum(jnp.where((lanes == 0) & (c0 == 0), 1, 0))
                return tot + jnp.sum(acc) - corr

            total = lax.fori_loop(0, B, count_row, jnp.int32(0))
            n_mask = jnp.maximum(
                1,
                (jnp.float32(_MASK_RATIO)
                 * (total.astype(jnp.float32) * jnp.float32(1.0 / B))
                 ).astype(jnp.int32))

            pltpu.sync_copy(pad_hbm.at[pl.ds(row * N, N)], pown_v)
            pltpu.sync_copy(sidx_hbm.at[pl.ds(row * N, N)], sidx_v)
            pltpu.sync_copy(sbits_hbm.at[pl.ds(row * N, N)], sbits_v)
            pltpu.sync_copy(bits_hbm.at[pl.ds(row * N, N)], bits_v)

            ones_m = jnp.ones((_L,), jnp.bool_)

            def wcond(carry):
                j, cum, _ = carry
                return (j < NCH) & (cum < n_mask)

            def wbody(carry):
                j, cum, thr = carry
                idxv = sidx_v[pl.ds(j * _L, _L)]
                pv = plsc.load_gather(pown_v, [idxv], mask=ones_m)
                elig = (pv == 0) & (idxv != 0)
                e = jnp.where(elig, 1, 0)
                cs = plsc.cumsum(e) + e
                crossm = (cum + cs == n_mask) & elig
                sb = sbits_v[pl.ds(j * _L, _L)]
                val = jnp.sum(jnp.where(crossm, sb, 0))
                found = jnp.sum(jnp.where(crossm, 1, 0)) > 0
                return (j + 1, cum + jnp.sum(e),
                        jnp.where(found, val, thr))

            _, _, thr = lax.while_loop(
                wcond, wbody,
                (jnp.int32(0), jnp.int32(0), jnp.int32(_ONE_BITS)))

            def mbody(i, _):
                v = bits_v[pl.ds(i * _L, _L)]
                p = pown_v[pl.ds(i * _L, _L)]
                pos = i * _L + lanes
                elig = (p == 0) & (pos != 0)
                rv = jnp.where(elig, v, _ONE_BITS)
                mask_v[pl.ds(i * _L, _L)] = jnp.where(rv <= thr, 1, 0)
                return 0

            lax.fori_loop(0, NCH, mbody, 0)
            pltpu.sync_copy(mask_v, mask_hbm.at[pl.ds(row * N, N)])

    return sel


def _apply_kernel(tok_ref, mask_ref, mt_ref, out_ref):
    mask = mask_ref[...]
    tok = tok_ref[...]
    mt = mt_ref[...]
    out_ref[...] = jnp.where(mask, mt[:, None, :], tok)


def kernel(tokens, padding_mask, mask_token):
    B, N, D = tokens.shape
    bits, sidx, sbits = _fixed_tables(B, N)
    pad_flat = padding_mask.astype(jnp.int32).reshape(B * N)

    sel = _build_sc_select(B, N)
    mask_i32 = sel(pad_flat, sidx, sbits, bits.reshape(-1))
    mask_out = mask_i32.reshape(B, N).astype(jnp.bool_)
    mask_sub = mask_out.reshape(B, N, 1)

    C = 2048
    grid = (B, N // C)
    out = pl.pallas_call(
        _apply_kernel,
        grid=grid,
        in_specs=[
            pl.BlockSpec((1, C, D), lambda b, c: (b, c, 0)),
            pl.BlockSpec((1, C, 1), lambda b, c: (b, c, 0)),
            pl.BlockSpec((1, D), lambda b, c: (0, 0)),
        ],
        out_specs=pl.BlockSpec((1, C, D), lambda b, c: (b, c, 0)),
        out_shape=jax.ShapeDtypeStruct((B, N, D), tokens.dtype),
    )(tokens, mask_sub, mask_token.reshape(1, D))

    return (out, mask_out)


# speedup vs baseline: 1.5079x; 1.5079x over previous
import jax
import jax.numpy as jnp
import numpy as np
from jax.experimental import pallas as pl

_MASK_RATIO = 0.15
_ONE_BITS = 0x3F800000

_RAND_CACHE = {}


def _fixed_rand_bits(B, N):
    if (B, N) not in _RAND_CACHE:
        with jax.ensure_compile_time_eval():
            r = np.asarray(
                jax.random.uniform(jax.random.key(42), (B, N), dtype=jnp.float32))
        _RAND_CACHE[(B, N)] = r.view(np.int32)
    return _RAND_CACHE[(B, N)]


def _select_kernel(bits_ref, pad_ref, mask_ref):
    bits = bits_ref[...]
    pad = pad_ref[...]
    B, N = bits.shape
    col = jax.lax.broadcasted_iota(jnp.int32, (B, N), 1)
    eligible = (col != 0) & jnp.logical_not(pad)
    total = jnp.sum(eligible.astype(jnp.float32))
    n_mask = jnp.maximum(1, (_MASK_RATIO * (total / B)).astype(jnp.int32))
    rv = jnp.where(eligible, bits, _ONE_BITS)

    lo0 = jnp.full((B, 1), -1, jnp.int32)
    hi0 = jnp.full((B, 1), _ONE_BITS, jnp.int32)

    def body(_, carry):
        lo, hi = carry
        mid = lo + (hi - lo) // 2
        cnt = jnp.sum((rv <= mid).astype(jnp.int32), axis=1, keepdims=True)
        ge = cnt >= n_mask
        return jnp.where(ge, lo, mid), jnp.where(ge, mid, hi)

    _, hi = jax.lax.fori_loop(0, 31, body, (lo0, hi0))
    mask_ref[...] = rv <= hi


def _apply_kernel(tok_ref, mask_ref, mt_ref, out_ref):
    mask = mask_ref[...]
    tok = tok_ref[...]
    mt = mt_ref[...]
    out_ref[...] = jnp.where(mask, mt[:, None, :], tok)


def kernel(tokens, padding_mask, mask_token):
    B, N, D = tokens.shape
    bits = _fixed_rand_bits(B, N)

    mask_out = pl.pallas_call(
        _select_kernel,
        out_shape=jax.ShapeDtypeStruct((B, N), jnp.bool_),
        out_specs=pl.BlockSpec((B, N), lambda: (0, 0)),
    )(bits, padding_mask)
    mask_sub = mask_out.reshape(B, N, 1)

    C = 4096
    grid = (B, N // C)
    out = pl.pallas_call(
        _apply_kernel,
        grid=grid,
        in_specs=[
            pl.BlockSpec((1, C, D), lambda b, c: (b, c, 0)),
            pl.BlockSpec((1, C, 1), lambda b, c: (b, c, 0)),
            pl.BlockSpec((1, D), lambda b, c: (0, 0)),
        ],
        out_specs=pl.BlockSpec((1, C, D), lambda b, c: (b, c, 0)),
        out_shape=jax.ShapeDtypeStruct((B, N, D), tokens.dtype),
    )(tokens, mask_sub, mask_token.reshape(1, D))

    return (out, mask_out)


# speedup vs baseline: 1.5277x; 1.0131x over previous
import jax
import jax.numpy as jnp
import numpy as np
from jax.experimental import pallas as pl

_MASK_RATIO = 0.15
_ONE_M = 1 << 23

_RAND_CACHE = {}


def _fixed_rand_bits(B, N):
    if (B, N) not in _RAND_CACHE:
        with jax.ensure_compile_time_eval():
            r = np.asarray(
                jax.random.uniform(jax.random.key(42), (B, N), dtype=jnp.float32))
        _RAND_CACHE[(B, N)] = (r.astype(np.float64) * _ONE_M).astype(np.int32)
    return _RAND_CACHE[(B, N)]


def _select_kernel(bits_ref, pad_ref, mask_ref):
    bits = bits_ref[...]
    pad = pad_ref[...]
    B, N = bits.shape
    col = jax.lax.broadcasted_iota(jnp.int32, (B, N), 1)
    eligible = (col != 0) & jnp.logical_not(pad)
    total = jnp.sum(eligible.astype(jnp.float32))
    n_mask = jnp.maximum(1, (_MASK_RATIO * (total / B)).astype(jnp.int32))
    rv = jnp.where(eligible, bits, _ONE_M)

    lo0 = jnp.full((B, 1), -1, jnp.int32)
    hi0 = jnp.full((B, 1), _ONE_M, jnp.int32)

    def body(_, carry):
        lo, hi = carry
        mid = lo + (hi - lo) // 2
        cnt = jnp.sum((rv <= mid).astype(jnp.int32), axis=1, keepdims=True)
        ge = cnt >= n_mask
        return jnp.where(ge, lo, mid), jnp.where(ge, mid, hi)

    _, hi = jax.lax.fori_loop(0, 24, body, (lo0, hi0))
    mask_ref[...] = rv <= hi


def _apply_kernel(tok_ref, mask_ref, mt_ref, out_ref):
    mask = mask_ref[...]
    tok = tok_ref[...]
    mt = mt_ref[...]
    out_ref[...] = jnp.where(mask, mt[:, None, :], tok)


def kernel(tokens, padding_mask, mask_token):
    B, N, D = tokens.shape
    bits = _fixed_rand_bits(B, N)

    mask_out = pl.pallas_call(
        _select_kernel,
        out_shape=jax.ShapeDtypeStruct((B, N), jnp.bool_),
        out_specs=pl.BlockSpec((B, N), lambda: (0, 0)),
    )(bits, padding_mask)
    mask_sub = mask_out.reshape(B, N, 1)

    C = 4096
    grid = (B, N // C)
    out = pl.pallas_call(
        _apply_kernel,
        grid=grid,
        in_specs=[
            pl.BlockSpec((1, C, D), lambda b, c: (b, c, 0)),
            pl.BlockSpec((1, C, 1), lambda b, c: (b, c, 0)),
            pl.BlockSpec((1, D), lambda b, c: (0, 0)),
        ],
        out_specs=pl.BlockSpec((1, C, D), lambda b, c: (b, c, 0)),
        out_shape=jax.ShapeDtypeStruct((B, N, D), tokens.dtype),
    )(tokens, mask_sub, mask_token.reshape(1, D))

    return (out, mask_out)
